# baseline (speedup 1.0000x reference)
import jax
import jax.numpy as jnp
import numpy as np
from jax import lax
from jax.experimental import pallas as pl
from jax.experimental.pallas import tpu as pltpu

_H = 12
_N = 32768
_A = 32
_K = 1024
_C = 2048
_NCHUNK = _N // _C
_INT_MIN = np.int32(-2147483648)


def _body(ret_ref, act_ref, mean_ref, std_ref, mask_ref, acc_ref, acc2_ref):
    j = pl.program_id(0)

    @pl.when(j == 0)
    def _build_mask():
        r = ret_ref[...]
        r = jnp.where(jnp.isnan(r), jnp.float32(0.0), r)
        bits = lax.bitcast_convert_type(r, jnp.int32)
        key = jnp.where(bits < 0, bits ^ np.int32(0x7FFFFFFF), bits)

        def bitstep(b, cand):
            test = cand | jnp.left_shift(np.int32(1), 31 - b)
            thresh = test ^ _INT_MIN
            cnt = jnp.sum((key >= thresh).astype(jnp.int32))
            return jnp.where(cnt >= _K, test, cand)

        cand = lax.fori_loop(0, 32, bitstep, jnp.int32(0))
        T = cand ^ _INT_MIN

        c_gt = jnp.sum((key > T).astype(jnp.int32))
        r_need = _K - c_gt
        eq = key == T
        idx = lax.broadcasted_iota(jnp.int32, (1, _N), 1)

        def jstep(b, jc):
            jt = jc | jnp.left_shift(np.int32(1), 15 - b)
            cnt = jnp.sum((eq & (idx < jt)).astype(jnp.int32))
            return jnp.where(cnt <= r_need, jt, jc)

        jstar = lax.fori_loop(0, 16, jstep, jnp.int32(0))

        sel = (key > T) | (eq & (idx < jstar))
        mask_ref[...] = sel.astype(jnp.float32)
        acc_ref[...] = jnp.zeros((_H, _A), jnp.float32)
        acc2_ref[...] = jnp.zeros((_H, _A), jnp.float32)

    m = mask_ref[:, pl.ds(j * _C, _C)]
    for h in range(_H):
        a = act_ref[h]
        s = jnp.dot(m, a, preferred_element_type=jnp.float32)
        s2 = jnp.dot(m, a * a, preferred_element_type=jnp.float32)
        acc_ref[h : h + 1, :] += s
        acc2_ref[h : h + 1, :] += s2

    @pl.when(j == _NCHUNK - 1)
    def _finalize():
        s = acc_ref[...]
        s2 = acc2_ref[...]
        mean = s * (1.0 / _K)
        var = jnp.maximum(s2 * (1.0 / _K) - mean * mean, 0.0)
        mean_ref[...] = mean.reshape(_H, 1, _A)
        std_ref[...] = jnp.sqrt(var).reshape(_H, 1, _A)


def kernel(actions, returns):
    out = pl.pallas_call(
        _body,
        grid=(_NCHUNK,),
        in_specs=[
            pl.BlockSpec((1, _N), lambda j: (0, 0)),
            pl.BlockSpec((_H, _C, _A), lambda j: (0, j, 0)),
        ],
        out_specs=[
            pl.BlockSpec((_H, 1, _A), lambda j: (0, 0, 0)),
            pl.BlockSpec((_H, 1, _A), lambda j: (0, 0, 0)),
        ],
        out_shape=[
            jax.ShapeDtypeStruct((_H, 1, _A), jnp.float32),
            jax.ShapeDtypeStruct((_H, 1, _A), jnp.float32),
        ],
        scratch_shapes=[
            pltpu.VMEM((1, _N), jnp.float32),
            pltpu.VMEM((_H, _A), jnp.float32),
            pltpu.VMEM((_H, _A), jnp.float32),
        ],
        compiler_params=pltpu.CompilerParams(
            dimension_semantics=("arbitrary",),
        ),
    )(returns.reshape(1, _N), actions)
    return (out[0], out[1])
